# baseline (speedup 1.0000x reference)
import functools

import jax
import jax.numpy as jnp
from jax import lax
from jax.experimental import pallas as pl
from jax.experimental.pallas import tpu as pltpu
from jax.experimental.pallas import tpu_sc as plsc

NC = 2
NS = 16
L = 16
NW = NC * NS

B, H, W = 32, 384, 384
RBLK = 24
NBLK = H // RBLK
SC_BLKS = 9
TC_BLKS = NBLK - SC_BLKS
VPR = W // L
NVEC = RBLK * VPR
UNIT = 4


def _sc_partials(p, t, m):
    mesh = plsc.VectorSubcoreMesh(core_axis_name="c", subcore_axis_name="s")

    @functools.partial(
        pl.kernel,
        mesh=mesh,
        out_type=(
            jax.ShapeDtypeStruct((NW, L), jnp.float32),
            jax.ShapeDtypeStruct((NW, L), jnp.int32),
        ),
        scratch_types=[
            pltpu.VMEM((2, RBLK, W), jnp.float32),
            pltpu.VMEM((2, RBLK, W), jnp.float32),
            pltpu.VMEM((2, RBLK, W), jnp.int32),
            pltpu.VMEM((L,), jnp.float32),
            pltpu.VMEM((L,), jnp.int32),
            pltpu.SemaphoreType.DMA,
            pltpu.SemaphoreType.DMA,
        ],
    )
    def k(p_hbm, t_hbm, m_hbm, num_hbm, cnt_hbm,
          p_v, t_v, m_v, num_v, cnt_v, sem0, sem1):
        wid = lax.axis_index("s") * NC + lax.axis_index("c")
        sems = (sem0, sem1)

        def issue(ci):
            slot = ci % 2
            sl = pl.ds(ci * RBLK, RBLK)
            return (
                pltpu.async_copy(p_hbm.at[wid, sl], p_v.at[slot], sems[slot]),
                pltpu.async_copy(t_hbm.at[wid, sl], t_v.at[slot], sems[slot]),
                pltpu.async_copy(m_hbm.at[wid, sl], m_v.at[slot], sems[slot]),
            )

        def compute(slot, acc, cnt):
            pr, tr, mr = p_v.at[slot], t_v.at[slot], m_v.at[slot]
            zero = jnp.zeros((L,), jnp.float32)
            zeroi = jnp.zeros((L,), jnp.int32)
            carry0 = (acc, zero, zero, zero, cnt, zeroi, zeroi, zeroi)

            @plsc.parallel_loop(0, NVEC, step=UNIT, unroll=2, carry=carry0)
            def body(i, c):
                a = list(c[:UNIT])
                n = list(c[UNIT:])
                r = i // VPR
                c0 = (i - r * VPR) * L
                for u in range(UNIT):
                    sl = pl.ds(c0 + u * L, L)
                    ad = jnp.abs(pr[r, sl] - tr[r, sl])
                    mv = mr[r, sl]
                    a[u] = a[u] + jnp.where(mv != 0, ad, 0.0)
                    n[u] = n[u] + mv
                return tuple(a) + tuple(n)

            c = body
            return (c[0] + c[1]) + (c[2] + c[3]), (c[4] + c[5]) + (c[6] + c[7])

        acc = jnp.zeros((L,), jnp.float32)
        cnt = jnp.zeros((L,), jnp.int32)
        handles = {0: issue(0)}
        for ci in range(SC_BLKS):
            if ci + 1 < SC_BLKS:
                handles[ci + 1] = issue(ci + 1)
            for h in handles.pop(ci):
                h.wait()
            acc, cnt = compute(ci % 2, acc, cnt)
        num_v[...] = acc
        cnt_v[...] = cnt
        pltpu.sync_copy(num_v, num_hbm.at[wid])
        pltpu.sync_copy(cnt_v, cnt_hbm.at[wid])

    return k(p, t, m)


def _tc_body(p_ref, t_ref, m_ref, num_ref, cnt_ref, acc_ref, cac_ref):
    i = pl.program_id(0)
    j = pl.program_id(1)

    @pl.when((i == 0) & (j == 0))
    def _init():
        acc_ref[...] = jnp.zeros_like(acc_ref)
        cac_ref[...] = jnp.zeros_like(cac_ref)

    pv = p_ref[0]
    tv = t_ref[0]
    mv = m_ref[0]
    ad = jnp.abs(pv - tv)
    acc_ref[...] += jnp.where(mv != 0, ad, 0.0)
    cac_ref[...] += mv

    @pl.when((i == B - 1) & (j == TC_BLKS - 1))
    def _fin():
        num_ref[...] = jnp.sum(acc_ref[...]).reshape(1, 1)
        cnt_ref[...] = jnp.sum(cac_ref[...]).reshape(1, 1)


def _tc_partials(p, t, m):
    in_spec = pl.BlockSpec((1, RBLK, W), lambda i, j: (i, SC_BLKS + j, 0))
    out_spec = pl.BlockSpec((1, 1), lambda i, j: (0, 0))
    return pl.pallas_call(
        _tc_body,
        grid=(B, TC_BLKS),
        in_specs=[in_spec, in_spec, in_spec],
        out_specs=[out_spec, out_spec],
        out_shape=[
            jax.ShapeDtypeStruct((1, 1), jnp.float32),
            jax.ShapeDtypeStruct((1, 1), jnp.int32),
        ],
        scratch_shapes=[
            pltpu.VMEM((RBLK, W), jnp.float32),
            pltpu.VMEM((RBLK, W), jnp.int32),
        ],
    )(p, t, m)


def _finish_body(nsc_ref, csc_ref, ntc_ref, ctc_ref, out_ref):
    s = jnp.sum(nsc_ref[...]) + ntc_ref[0, 0]
    c = jnp.sum(csc_ref[...]) + ctc_ref[0, 0]
    out_ref[...] = (s / (2.0 * c.astype(jnp.float32))).reshape(1, 1)


def kernel(prediction, target, mask):
    num_sc, cnt_sc = _sc_partials(prediction, target, mask)
    num_tc, cnt_tc = _tc_partials(prediction, target, mask)
    out = pl.pallas_call(
        _finish_body,
        out_shape=jax.ShapeDtypeStruct((1, 1), jnp.float32),
    )(num_sc, cnt_sc, num_tc, cnt_tc)
    return out[0, 0]


# speedup vs baseline: 2.1637x; 2.1637x over previous
import functools

import jax
import jax.numpy as jnp
from jax import lax
from jax.experimental import pallas as pl
from jax.experimental.pallas import tpu as pltpu
from jax.experimental.pallas import tpu_sc as plsc

NC = 2
NS = 16
L = 16
NW = NC * NS

B, H, W = 32, 384, 384
RBLK = 24
SC_BLKS = 8
TC_RBLK = 96
TC_BLK0 = SC_BLKS * RBLK // TC_RBLK
TC_BLKS = H // TC_RBLK - TC_BLK0
VPR = W // L
NVEC = RBLK * VPR
UNIT = 4


def _sc_partials(p, t, m):
    mesh = plsc.VectorSubcoreMesh(core_axis_name="c", subcore_axis_name="s")

    @functools.partial(
        pl.kernel,
        mesh=mesh,
        out_type=(
            jax.ShapeDtypeStruct((NW, L), jnp.float32),
            jax.ShapeDtypeStruct((NW, L), jnp.int32),
        ),
        scratch_types=[
            pltpu.VMEM((2, RBLK, W), jnp.float32),
            pltpu.VMEM((2, RBLK, W), jnp.float32),
            pltpu.VMEM((2, RBLK, W), jnp.int32),
            pltpu.VMEM((L,), jnp.float32),
            pltpu.VMEM((L,), jnp.int32),
            pltpu.SemaphoreType.DMA,
            pltpu.SemaphoreType.DMA,
        ],
    )
    def k(p_hbm, t_hbm, m_hbm, num_hbm, cnt_hbm,
          p_v, t_v, m_v, num_v, cnt_v, sem0, sem1):
        wid = lax.axis_index("s") * NC + lax.axis_index("c")
        sems = (sem0, sem1)

        def issue(ci):
            slot = ci % 2
            sl = pl.ds(ci * RBLK, RBLK)
            return (
                pltpu.async_copy(p_hbm.at[wid, sl], p_v.at[slot], sems[slot]),
                pltpu.async_copy(t_hbm.at[wid, sl], t_v.at[slot], sems[slot]),
                pltpu.async_copy(m_hbm.at[wid, sl], m_v.at[slot], sems[slot]),
            )

        def compute(slot, acc, cnt):
            pr, tr, mr = p_v.at[slot], t_v.at[slot], m_v.at[slot]
            zero = jnp.zeros((L,), jnp.float32)
            zeroi = jnp.zeros((L,), jnp.int32)
            carry0 = (acc, zero, zero, zero, cnt, zeroi, zeroi, zeroi)

            @plsc.parallel_loop(0, NVEC, step=UNIT, unroll=2, carry=carry0)
            def body(i, c):
                a = list(c[:UNIT])
                n = list(c[UNIT:])
                r = i // VPR
                c0 = (i - r * VPR) * L
                for u in range(UNIT):
                    sl = pl.ds(c0 + u * L, L)
                    ad = jnp.abs(pr[r, sl] - tr[r, sl])
                    mv = mr[r, sl]
                    a[u] = a[u] + jnp.where(mv != 0, ad, 0.0)
                    n[u] = n[u] + mv
                return tuple(a) + tuple(n)

            c = body
            return (c[0] + c[1]) + (c[2] + c[3]), (c[4] + c[5]) + (c[6] + c[7])

        acc = jnp.zeros((L,), jnp.float32)
        cnt = jnp.zeros((L,), jnp.int32)
        handles = {0: issue(0)}
        for ci in range(SC_BLKS):
            if ci + 1 < SC_BLKS:
                handles[ci + 1] = issue(ci + 1)
            for h in handles.pop(ci):
                h.wait()
            acc, cnt = compute(ci % 2, acc, cnt)
        num_v[...] = acc
        cnt_v[...] = cnt
        pltpu.sync_copy(num_v, num_hbm.at[wid])
        pltpu.sync_copy(cnt_v, cnt_hbm.at[wid])

    return k(p, t, m)


def _tc_body(p_ref, t_ref, m_ref, num_ref, cnt_ref, acc_ref, cac_ref):
    i = pl.program_id(0)
    j = pl.program_id(1)

    @pl.when((i == 0) & (j == 0))
    def _init():
        acc_ref[...] = jnp.zeros_like(acc_ref)
        cac_ref[...] = jnp.zeros_like(cac_ref)

    pv = p_ref[0]
    tv = t_ref[0]
    mv = m_ref[0]
    ad = jnp.abs(pv - tv)
    acc_ref[...] += jnp.where(mv != 0, ad, 0.0)
    cac_ref[...] += mv

    @pl.when((i == B - 1) & (j == TC_BLKS - 1))
    def _fin():
        num_ref[...] = jnp.sum(acc_ref[...]).reshape(1, 1)
        cnt_ref[...] = jnp.sum(cac_ref[...]).reshape(1, 1)


def _tc_partials(p, t, m):
    in_spec = pl.BlockSpec((1, TC_RBLK, W), lambda i, j: (i, TC_BLK0 + j, 0))
    out_spec = pl.BlockSpec((1, 1), lambda i, j: (0, 0))
    return pl.pallas_call(
        _tc_body,
        grid=(B, TC_BLKS),
        in_specs=[in_spec, in_spec, in_spec],
        out_specs=[out_spec, out_spec],
        out_shape=[
            jax.ShapeDtypeStruct((1, 1), jnp.float32),
            jax.ShapeDtypeStruct((1, 1), jnp.int32),
        ],
        scratch_shapes=[
            pltpu.VMEM((TC_RBLK, W), jnp.float32),
            pltpu.VMEM((TC_RBLK, W), jnp.int32),
        ],
    )(p, t, m)


def _finish_body(nsc_ref, csc_ref, ntc_ref, ctc_ref, out_ref):
    s = jnp.sum(nsc_ref[...]) + ntc_ref[0, 0]
    c = jnp.sum(csc_ref[...]) + ctc_ref[0, 0]
    out_ref[...] = (s / (2.0 * c.astype(jnp.float32))).reshape(1, 1)


def kernel(prediction, target, mask):
    num_sc, cnt_sc = _sc_partials(prediction, target, mask)
    num_tc, cnt_tc = _tc_partials(prediction, target, mask)
    out = pl.pallas_call(
        _finish_body,
        out_shape=jax.ShapeDtypeStruct((1, 1), jnp.float32),
    )(num_sc, cnt_sc, num_tc, cnt_tc)
    return out[0, 0]


# speedup vs baseline: 3.2131x; 1.4850x over previous
import functools

import jax
import jax.numpy as jnp
from jax import lax
from jax.experimental import pallas as pl
from jax.experimental.pallas import tpu as pltpu
from jax.experimental.pallas import tpu_sc as plsc

NC = 2
NS = 16
L = 16
NW = NC * NS

B, H, W = 32, 384, 384
RBLK = 24
SC_BLKS = 8
TC_ROWS = H - SC_BLKS * RBLK
TC_IMGS = 4
VPR = W // L
NVEC = RBLK * VPR
UNIT = 4


def _sc_partials(p, t, m):
    mesh = plsc.VectorSubcoreMesh(core_axis_name="c", subcore_axis_name="s")

    @functools.partial(
        pl.kernel,
        mesh=mesh,
        out_type=(
            jax.ShapeDtypeStruct((NW, L), jnp.float32),
            jax.ShapeDtypeStruct((NW, L), jnp.int32),
        ),
        scratch_types=[
            pltpu.VMEM((2, RBLK, W), jnp.float32),
            pltpu.VMEM((2, RBLK, W), jnp.float32),
            pltpu.VMEM((2, RBLK, W), jnp.int32),
            pltpu.VMEM((L,), jnp.float32),
            pltpu.VMEM((L,), jnp.int32),
            pltpu.SemaphoreType.DMA,
            pltpu.SemaphoreType.DMA,
        ],
    )
    def k(p_hbm, t_hbm, m_hbm, num_hbm, cnt_hbm,
          p_v, t_v, m_v, num_v, cnt_v, sem0, sem1):
        wid = lax.axis_index("s") * NC + lax.axis_index("c")
        sems = (sem0, sem1)

        def issue(ci):
            slot = ci % 2
            sl = pl.ds(ci * RBLK, RBLK)
            return (
                pltpu.async_copy(p_hbm.at[wid, sl], p_v.at[slot], sems[slot]),
                pltpu.async_copy(t_hbm.at[wid, sl], t_v.at[slot], sems[slot]),
                pltpu.async_copy(m_hbm.at[wid, sl], m_v.at[slot], sems[slot]),
            )

        def compute(slot, acc, cnt):
            pr, tr, mr = p_v.at[slot], t_v.at[slot], m_v.at[slot]
            zero = jnp.zeros((L,), jnp.float32)
            zeroi = jnp.zeros((L,), jnp.int32)
            carry0 = (acc, zero, zero, zero, cnt, zeroi, zeroi, zeroi)

            @plsc.parallel_loop(0, NVEC, step=UNIT, unroll=2, carry=carry0)
            def body(i, c):
                a = list(c[:UNIT])
                n = list(c[UNIT:])
                r = i // VPR
                c0 = (i - r * VPR) * L
                for u in range(UNIT):
                    sl = pl.ds(c0 + u * L, L)
                    ad = jnp.abs(pr[r, sl] - tr[r, sl])
                    mv = mr[r, sl]
                    a[u] = a[u] + jnp.where(mv != 0, ad, 0.0)
                    n[u] = n[u] + mv
                return tuple(a) + tuple(n)

            c = body
            return (c[0] + c[1]) + (c[2] + c[3]), (c[4] + c[5]) + (c[6] + c[7])

        acc = jnp.zeros((L,), jnp.float32)
        cnt = jnp.zeros((L,), jnp.int32)
        handles = {0: issue(0)}
        for ci in range(SC_BLKS):
            if ci + 1 < SC_BLKS:
                handles[ci + 1] = issue(ci + 1)
            for h in handles.pop(ci):
                h.wait()
            acc, cnt = compute(ci % 2, acc, cnt)
        num_v[...] = acc
        cnt_v[...] = cnt
        pltpu.sync_copy(num_v, num_hbm.at[wid])
        pltpu.sync_copy(cnt_v, cnt_hbm.at[wid])

    return k(p, t, m)


def _tc_body(p_ref, t_ref, m_ref, num_ref, cnt_ref):
    pv = p_ref[...]
    tv = t_ref[...]
    mv = m_ref[...]
    ad = jnp.abs(pv - tv)
    num_ref[...] = jnp.sum(jnp.where(mv != 0, ad, 0.0)).reshape(1, 1, 1)
    cnt_ref[...] = jnp.sum(mv).reshape(1, 1, 1)


def _tc_partials(p, t, m):
    in_spec = pl.BlockSpec((TC_IMGS, TC_ROWS, W), lambda i: (i, 1, 0))
    out_spec = pl.BlockSpec((1, 1, 1), lambda i: (i, 0, 0))
    return pl.pallas_call(
        _tc_body,
        grid=(B // TC_IMGS,),
        in_specs=[in_spec, in_spec, in_spec],
        out_specs=[out_spec, out_spec],
        out_shape=[
            jax.ShapeDtypeStruct((B // TC_IMGS, 1, 1), jnp.float32),
            jax.ShapeDtypeStruct((B // TC_IMGS, 1, 1), jnp.int32),
        ],
    )(p, t, m)


def _finish_body(nsc_ref, csc_ref, ntc_ref, ctc_ref, out_ref):
    s = jnp.sum(nsc_ref[...]) + jnp.sum(ntc_ref[...])
    c = jnp.sum(csc_ref[...]) + jnp.sum(ctc_ref[...])
    out_ref[...] = (s / (2.0 * c.astype(jnp.float32))).reshape(1, 1)


def kernel(prediction, target, mask):
    num_sc, cnt_sc = _sc_partials(prediction, target, mask)
    num_tc, cnt_tc = _tc_partials(prediction, target, mask)
    out = pl.pallas_call(
        _finish_body,
        out_shape=jax.ShapeDtypeStruct((1, 1), jnp.float32),
    )(num_sc, cnt_sc, num_tc, cnt_tc)
    return out[0, 0]


# speedup vs baseline: 3.3151x; 1.0317x over previous
import functools

import jax
import jax.numpy as jnp
from jax import lax
from jax.experimental import pallas as pl
from jax.experimental.pallas import tpu as pltpu
from jax.experimental.pallas import tpu_sc as plsc

NC = 2
NS = 16
L = 16
NW = NC * NS

B, H, W = 32, 384, 384
RBLK = 24
SC_BLKS = 6
TC_ROWS = H - SC_BLKS * RBLK
SC_ROW0 = TC_ROWS
TC_IMGS = 4
VPR = W // L
NVEC = RBLK * VPR
UNIT = 4


def _sc_partials(p, t, m):
    mesh = plsc.VectorSubcoreMesh(core_axis_name="c", subcore_axis_name="s")

    @functools.partial(
        pl.kernel,
        mesh=mesh,
        out_type=(
            jax.ShapeDtypeStruct((NW, L), jnp.float32),
            jax.ShapeDtypeStruct((NW, L), jnp.int32),
        ),
        scratch_types=[
            pltpu.VMEM((2, RBLK, W), jnp.float32),
            pltpu.VMEM((2, RBLK, W), jnp.float32),
            pltpu.VMEM((2, RBLK, W), jnp.int32),
            pltpu.VMEM((L,), jnp.float32),
            pltpu.VMEM((L,), jnp.int32),
            pltpu.SemaphoreType.DMA,
            pltpu.SemaphoreType.DMA,
        ],
    )
    def k(p_hbm, t_hbm, m_hbm, num_hbm, cnt_hbm,
          p_v, t_v, m_v, num_v, cnt_v, sem0, sem1):
        wid = lax.axis_index("s") * NC + lax.axis_index("c")
        sems = (sem0, sem1)

        def issue(ci):
            slot = ci % 2
            sl = pl.ds(SC_ROW0 + ci * RBLK, RBLK)
            return (
                pltpu.async_copy(p_hbm.at[wid, sl], p_v.at[slot], sems[slot]),
                pltpu.async_copy(t_hbm.at[wid, sl], t_v.at[slot], sems[slot]),
                pltpu.async_copy(m_hbm.at[wid, sl], m_v.at[slot], sems[slot]),
            )

        def compute(slot, acc, cnt):
            pr, tr, mr = p_v.at[slot], t_v.at[slot], m_v.at[slot]
            zero = jnp.zeros((L,), jnp.float32)
            zeroi = jnp.zeros((L,), jnp.int32)
            carry0 = (acc, zero, zero, zero, cnt, zeroi, zeroi, zeroi)

            @plsc.parallel_loop(0, NVEC, step=UNIT, unroll=2, carry=carry0)
            def body(i, c):
                a = list(c[:UNIT])
                n = list(c[UNIT:])
                r = i // VPR
                c0 = (i - r * VPR) * L
                for u in range(UNIT):
                    sl = pl.ds(c0 + u * L, L)
                    ad = jnp.abs(pr[r, sl] - tr[r, sl])
                    mv = mr[r, sl]
                    a[u] = a[u] + jnp.where(mv != 0, ad, 0.0)
                    n[u] = n[u] + mv
                return tuple(a) + tuple(n)

            c = body
            return (c[0] + c[1]) + (c[2] + c[3]), (c[4] + c[5]) + (c[6] + c[7])

        acc = jnp.zeros((L,), jnp.float32)
        cnt = jnp.zeros((L,), jnp.int32)
        handles = {0: issue(0)}
        for ci in range(SC_BLKS):
            if ci + 1 < SC_BLKS:
                handles[ci + 1] = issue(ci + 1)
            for h in handles.pop(ci):
                h.wait()
            acc, cnt = compute(ci % 2, acc, cnt)
        num_v[...] = acc
        cnt_v[...] = cnt
        pltpu.sync_copy(num_v, num_hbm.at[wid])
        pltpu.sync_copy(cnt_v, cnt_hbm.at[wid])

    return k(p, t, m)


def _tc_body(p_ref, t_ref, m_ref, num_ref, cnt_ref):
    pv = p_ref[...]
    tv = t_ref[...]
    mv = m_ref[...]
    ad = jnp.abs(pv - tv)
    num_ref[...] = jnp.sum(jnp.where(mv != 0, ad, 0.0)).reshape(1, 1, 1)
    cnt_ref[...] = jnp.sum(mv).reshape(1, 1, 1)


def _tc_partials(p, t, m):
    in_spec = pl.BlockSpec((TC_IMGS, TC_ROWS, W), lambda i: (i, 0, 0))
    out_spec = pl.BlockSpec((1, 1, 1), lambda i: (i, 0, 0))
    return pl.pallas_call(
        _tc_body,
        grid=(B // TC_IMGS,),
        in_specs=[in_spec, in_spec, in_spec],
        out_specs=[out_spec, out_spec],
        out_shape=[
            jax.ShapeDtypeStruct((B // TC_IMGS, 1, 1), jnp.float32),
            jax.ShapeDtypeStruct((B // TC_IMGS, 1, 1), jnp.int32),
        ],
    )(p, t, m)


def _finish_body(nsc_ref, csc_ref, ntc_ref, ctc_ref, out_ref):
    s = jnp.sum(nsc_ref[...]) + jnp.sum(ntc_ref[...])
    c = jnp.sum(csc_ref[...]) + jnp.sum(ctc_ref[...])
    out_ref[...] = (s / (2.0 * c.astype(jnp.float32))).reshape(1, 1)


def kernel(prediction, target, mask):
    num_sc, cnt_sc = _sc_partials(prediction, target, mask)
    num_tc, cnt_tc = _tc_partials(prediction, target, mask)
    out = pl.pallas_call(
        _finish_body,
        out_shape=jax.ShapeDtypeStruct((1, 1), jnp.float32),
    )(num_sc, cnt_sc, num_tc, cnt_tc)
    return out[0, 0]


# speedup vs baseline: 3.3381x; 1.0070x over previous
import functools

import jax
import jax.numpy as jnp
from jax import lax
from jax.experimental import pallas as pl
from jax.experimental.pallas import tpu as pltpu
from jax.experimental.pallas import tpu_sc as plsc

NC = 2
NS = 16
L = 16
NW = NC * NS

B, H, W = 32, 384, 384
RBLK = 24
SC_BLKS = 6
TC_ROWS = H - SC_BLKS * RBLK
SC_ROW0 = TC_ROWS
TC_IMGS = 8
VPR = W // L
NVEC = RBLK * VPR
UNIT = 4


def _sc_partials(p, t, m):
    mesh = plsc.VectorSubcoreMesh(core_axis_name="c", subcore_axis_name="s")

    @functools.partial(
        pl.kernel,
        mesh=mesh,
        out_type=(
            jax.ShapeDtypeStruct((NW, L), jnp.float32),
            jax.ShapeDtypeStruct((NW, L), jnp.int32),
        ),
        scratch_types=[
            pltpu.VMEM((2, RBLK, W), jnp.float32),
            pltpu.VMEM((2, RBLK, W), jnp.float32),
            pltpu.VMEM((2, RBLK, W), jnp.int32),
            pltpu.VMEM((L,), jnp.float32),
            pltpu.VMEM((L,), jnp.int32),
            pltpu.SemaphoreType.DMA,
            pltpu.SemaphoreType.DMA,
        ],
    )
    def k(p_hbm, t_hbm, m_hbm, num_hbm, cnt_hbm,
          p_v, t_v, m_v, num_v, cnt_v, sem0, sem1):
        wid = lax.axis_index("s") * NC + lax.axis_index("c")
        sems = (sem0, sem1)

        def issue(ci):
            slot = ci % 2
            sl = pl.ds(SC_ROW0 + ci * RBLK, RBLK)
            return (
                pltpu.async_copy(p_hbm.at[wid, sl], p_v.at[slot], sems[slot]),
                pltpu.async_copy(t_hbm.at[wid, sl], t_v.at[slot], sems[slot]),
                pltpu.async_copy(m_hbm.at[wid, sl], m_v.at[slot], sems[slot]),
            )

        def compute(slot, acc, cnt):
            pr, tr, mr = p_v.at[slot], t_v.at[slot], m_v.at[slot]
            zero = jnp.zeros((L,), jnp.float32)
            zeroi = jnp.zeros((L,), jnp.int32)
            carry0 = (acc, zero, zero, zero, cnt, zeroi, zeroi, zeroi)

            @plsc.parallel_loop(0, NVEC, step=UNIT, unroll=2, carry=carry0)
            def body(i, c):
                a = list(c[:UNIT])
                n = list(c[UNIT:])
                r = i // VPR
                c0 = (i - r * VPR) * L
                for u in range(UNIT):
                    sl = pl.ds(c0 + u * L, L)
                    ad = jnp.abs(pr[r, sl] - tr[r, sl])
                    mv = mr[r, sl]
                    a[u] = a[u] + jnp.where(mv != 0, ad, 0.0)
                    n[u] = n[u] + mv
                return tuple(a) + tuple(n)

            c = body
            return (c[0] + c[1]) + (c[2] + c[3]), (c[4] + c[5]) + (c[6] + c[7])

        acc = jnp.zeros((L,), jnp.float32)
        cnt = jnp.zeros((L,), jnp.int32)
        handles = {0: issue(0)}
        for ci in range(SC_BLKS):
            if ci + 1 < SC_BLKS:
                handles[ci + 1] = issue(ci + 1)
            for h in handles.pop(ci):
                h.wait()
            acc, cnt = compute(ci % 2, acc, cnt)
        num_v[...] = acc
        cnt_v[...] = cnt
        pltpu.sync_copy(num_v, num_hbm.at[wid])
        pltpu.sync_copy(cnt_v, cnt_hbm.at[wid])

    return k(p, t, m)


def _tc_body(p_ref, t_ref, m_ref, num_ref, cnt_ref):
    pv = p_ref[...]
    tv = t_ref[...]
    mv = m_ref[...]
    ad = jnp.abs(pv - tv)
    num_ref[...] = jnp.sum(jnp.where(mv != 0, ad, 0.0)).reshape(1, 1, 1)
    cnt_ref[...] = jnp.sum(mv).reshape(1, 1, 1)


def _tc_partials(p, t, m):
    in_spec = pl.BlockSpec((TC_IMGS, TC_ROWS, W), lambda i: (i, 0, 0))
    out_spec = pl.BlockSpec((1, 1, 1), lambda i: (i, 0, 0))
    return pl.pallas_call(
        _tc_body,
        grid=(B // TC_IMGS,),
        in_specs=[in_spec, in_spec, in_spec],
        out_specs=[out_spec, out_spec],
        out_shape=[
            jax.ShapeDtypeStruct((B // TC_IMGS, 1, 1), jnp.float32),
            jax.ShapeDtypeStruct((B // TC_IMGS, 1, 1), jnp.int32),
        ],
    )(p, t, m)


def _finish_body(nsc_ref, csc_ref, ntc_ref, ctc_ref, out_ref):
    s = jnp.sum(nsc_ref[...]) + jnp.sum(ntc_ref[...])
    c = jnp.sum(csc_ref[...]) + jnp.sum(ctc_ref[...])
    out_ref[...] = (s / (2.0 * c.astype(jnp.float32))).reshape(1, 1)


def kernel(prediction, target, mask):
    num_sc, cnt_sc = _sc_partials(prediction, target, mask)
    num_tc, cnt_tc = _tc_partials(prediction, target, mask)
    out = pl.pallas_call(
        _finish_body,
        out_shape=jax.ShapeDtypeStruct((1, 1), jnp.float32),
    )(num_sc, cnt_sc, num_tc, cnt_tc)
    return out[0, 0]


# speedup vs baseline: 3.4329x; 1.0284x over previous
import functools

import jax
import jax.numpy as jnp
from jax import lax
from jax.experimental import pallas as pl
from jax.experimental.pallas import tpu as pltpu
from jax.experimental.pallas import tpu_sc as plsc

NC = 2
NS = 16
L = 16
NW = NC * NS

B, H, W = 32, 384, 384
TC_ROWS = 256
SC_ROW0 = TC_ROWS
NCHUNK = 8
CH_ROWS = (H - SC_ROW0) // NCHUNK
TC_IMGS = 8
VPR = W // L
NVEC = CH_ROWS * VPR
UNIT = 4


def _sc_partials(p, t, m):
    mesh = plsc.VectorSubcoreMesh(core_axis_name="c", subcore_axis_name="s")

    @functools.partial(
        pl.kernel,
        mesh=mesh,
        out_type=(
            jax.ShapeDtypeStruct((NW, L), jnp.float32),
            jax.ShapeDtypeStruct((NW, L), jnp.int32),
        ),
        scratch_types=[
            pltpu.VMEM((2, CH_ROWS, W), jnp.float32),
            pltpu.VMEM((2, CH_ROWS, W), jnp.float32),
            pltpu.VMEM((2, CH_ROWS, W), jnp.int32),
            pltpu.VMEM((L,), jnp.float32),
            pltpu.VMEM((L,), jnp.int32),
            pltpu.SemaphoreType.DMA,
            pltpu.SemaphoreType.DMA,
        ],
    )
    def k(p_hbm, t_hbm, m_hbm, num_hbm, cnt_hbm,
          p_v, t_v, m_v, num_v, cnt_v, sem0, sem1):
        wid = lax.axis_index("s") * NC + lax.axis_index("c")
        sems = (sem0, sem1)

        def issue(ci):
            slot = ci % 2
            sl = pl.ds(SC_ROW0 + ci * CH_ROWS, CH_ROWS)
            return (
                pltpu.async_copy(p_hbm.at[wid, sl], p_v.at[slot], sems[slot]),
                pltpu.async_copy(t_hbm.at[wid, sl], t_v.at[slot], sems[slot]),
                pltpu.async_copy(m_hbm.at[wid, sl], m_v.at[slot], sems[slot]),
            )

        def compute(slot, acc, cnt):
            pr, tr, mr = p_v.at[slot], t_v.at[slot], m_v.at[slot]
            zero = jnp.zeros((L,), jnp.float32)
            zeroi = jnp.zeros((L,), jnp.int32)
            carry0 = (acc, zero, zero, zero, cnt, zeroi, zeroi, zeroi)

            @plsc.parallel_loop(0, NVEC, step=UNIT, unroll=2, carry=carry0)
            def body(i, c):
                a = list(c[:UNIT])
                n = list(c[UNIT:])
                r = i // VPR
                c0 = (i - r * VPR) * L
                for u in range(UNIT):
                    sl = pl.ds(c0 + u * L, L)
                    ad = jnp.abs(pr[r, sl] - tr[r, sl])
                    mv = mr[r, sl]
                    a[u] = a[u] + jnp.where(mv != 0, ad, 0.0)
                    n[u] = n[u] + mv
                return tuple(a) + tuple(n)

            c = body
            return (c[0] + c[1]) + (c[2] + c[3]), (c[4] + c[5]) + (c[6] + c[7])

        acc = jnp.zeros((L,), jnp.float32)
        cnt = jnp.zeros((L,), jnp.int32)
        handles = {0: issue(0)}
        for ci in range(NCHUNK):
            if ci + 1 < NCHUNK:
                handles[ci + 1] = issue(ci + 1)
            for h in handles.pop(ci):
                h.wait()
            acc, cnt = compute(ci % 2, acc, cnt)
        num_v[...] = acc
        cnt_v[...] = cnt
        pltpu.sync_copy(num_v, num_hbm.at[wid])
        pltpu.sync_copy(cnt_v, cnt_hbm.at[wid])

    return k(p, t, m)


def _tc_body(p_ref, t_ref, m_ref, num_ref, cnt_ref):
    pv = p_ref[...]
    tv = t_ref[...]
    mv = m_ref[...]
    ad = jnp.abs(pv - tv)
    num_ref[...] = jnp.sum(jnp.where(mv != 0, ad, 0.0)).reshape(1, 1, 1)
    cnt_ref[...] = jnp.sum(mv).reshape(1, 1, 1)


def _tc_partials(p, t, m):
    in_spec = pl.BlockSpec((TC_IMGS, TC_ROWS, W), lambda i: (i, 0, 0))
    out_spec = pl.BlockSpec((1, 1, 1), lambda i: (i, 0, 0))
    return pl.pallas_call(
        _tc_body,
        grid=(B // TC_IMGS,),
        in_specs=[in_spec, in_spec, in_spec],
        out_specs=[out_spec, out_spec],
        out_shape=[
            jax.ShapeDtypeStruct((B // TC_IMGS, 1, 1), jnp.float32),
            jax.ShapeDtypeStruct((B // TC_IMGS, 1, 1), jnp.int32),
        ],
    )(p, t, m)


def _finish_body(nsc_ref, csc_ref, ntc_ref, ctc_ref, out_ref):
    s = jnp.sum(nsc_ref[...]) + jnp.sum(ntc_ref[...])
    c = jnp.sum(csc_ref[...]) + jnp.sum(ctc_ref[...])
    out_ref[...] = (s / (2.0 * c.astype(jnp.float32))).reshape(1, 1)


def kernel(prediction, target, mask):
    num_sc, cnt_sc = _sc_partials(prediction, target, mask)
    num_tc, cnt_tc = _tc_partials(prediction, target, mask)
    out = pl.pallas_call(
        _finish_body,
        out_shape=jax.ShapeDtypeStruct((1, 1), jnp.float32),
    )(num_sc, cnt_sc, num_tc, cnt_tc)
    return out[0, 0]


# speedup vs baseline: 3.4348x; 1.0006x over previous
import functools

import jax
import jax.numpy as jnp
from jax import lax
from jax.experimental import pallas as pl
from jax.experimental.pallas import tpu as pltpu
from jax.experimental.pallas import tpu_sc as plsc

NC = 2
NS = 16
L = 16
NW = NC * NS

B, H, W = 32, 384, 384
TC_ROWS = 264
SC_ROW0 = TC_ROWS
NCHUNK = 5
CH_ROWS = (H - SC_ROW0) // NCHUNK
TC_IMGS = 8
VPR = W // L
NVEC = CH_ROWS * VPR
UNIT = 8


def _sc_partials(p, t, m):
    mesh = plsc.VectorSubcoreMesh(core_axis_name="c", subcore_axis_name="s")

    @functools.partial(
        pl.kernel,
        mesh=mesh,
        out_type=(
            jax.ShapeDtypeStruct((NW, L), jnp.float32),
            jax.ShapeDtypeStruct((NW, L), jnp.int32),
        ),
        scratch_types=[
            pltpu.VMEM((2, CH_ROWS, W), jnp.float32),
            pltpu.VMEM((2, CH_ROWS, W), jnp.float32),
            pltpu.VMEM((2, CH_ROWS, W), jnp.int32),
            pltpu.VMEM((L,), jnp.float32),
            pltpu.VMEM((L,), jnp.int32),
            pltpu.SemaphoreType.DMA,
            pltpu.SemaphoreType.DMA,
        ],
    )
    def k(p_hbm, t_hbm, m_hbm, num_hbm, cnt_hbm,
          p_v, t_v, m_v, num_v, cnt_v, sem0, sem1):
        wid = lax.axis_index("s") * NC + lax.axis_index("c")
        sems = (sem0, sem1)

        def issue(ci):
            slot = ci % 2
            sl = pl.ds(SC_ROW0 + ci * CH_ROWS, CH_ROWS)
            return (
                pltpu.async_copy(p_hbm.at[wid, sl], p_v.at[slot], sems[slot]),
                pltpu.async_copy(t_hbm.at[wid, sl], t_v.at[slot], sems[slot]),
                pltpu.async_copy(m_hbm.at[wid, sl], m_v.at[slot], sems[slot]),
            )

        def compute(slot, acc, cnt):
            pr, tr, mr = p_v.at[slot], t_v.at[slot], m_v.at[slot]
            zero = jnp.zeros((L,), jnp.float32)
            zeroi = jnp.zeros((L,), jnp.int32)
            carry0 = (acc,) + (zero,) * (UNIT - 1) + (cnt,) + (zeroi,) * (UNIT - 1)

            @plsc.parallel_loop(0, NVEC, step=UNIT, unroll=2, carry=carry0)
            def body(i, c):
                a = list(c[:UNIT])
                n = list(c[UNIT:])
                r = i // VPR
                c0 = (i - r * VPR) * L
                for u in range(UNIT):
                    sl = pl.ds(c0 + u * L, L)
                    ad = jnp.abs(pr[r, sl] - tr[r, sl])
                    mv = mr[r, sl]
                    a[u] = a[u] + jnp.where(mv != 0, ad, 0.0)
                    n[u] = n[u] + mv
                return tuple(a) + tuple(n)

            c = body

            def tree_sum(vals):
                vals = list(vals)
                while len(vals) > 1:
                    vals = [vals[i] + vals[i + 1] for i in range(0, len(vals) - 1, 2)] + (
                        [vals[-1]] if len(vals) % 2 else [])
                return vals[0]

            return tree_sum(c[:UNIT]), tree_sum(c[UNIT:])

        acc = jnp.zeros((L,), jnp.float32)
        cnt = jnp.zeros((L,), jnp.int32)
        handles = {0: issue(0)}
        for ci in range(NCHUNK):
            if ci + 1 < NCHUNK:
                handles[ci + 1] = issue(ci + 1)
            for h in handles.pop(ci):
                h.wait()
            acc, cnt = compute(ci % 2, acc, cnt)
        num_v[...] = acc
        cnt_v[...] = cnt
        pltpu.sync_copy(num_v, num_hbm.at[wid])
        pltpu.sync_copy(cnt_v, cnt_hbm.at[wid])

    return k(p, t, m)


def _tc_body(p_ref, t_ref, m_ref, num_ref, cnt_ref):
    pv = p_ref[...]
    tv = t_ref[...]
    mv = m_ref[...]
    ad = jnp.abs(pv - tv)
    num_ref[...] = jnp.sum(jnp.where(mv != 0, ad, 0.0)).reshape(1, 1, 1)
    cnt_ref[...] = jnp.sum(mv).reshape(1, 1, 1)


def _tc_partials(p, t, m):
    in_spec = pl.BlockSpec((TC_IMGS, TC_ROWS, W), lambda i: (i, 0, 0))
    out_spec = pl.BlockSpec((1, 1, 1), lambda i: (i, 0, 0))
    return pl.pallas_call(
        _tc_body,
        grid=(B // TC_IMGS,),
        in_specs=[in_spec, in_spec, in_spec],
        out_specs=[out_spec, out_spec],
        out_shape=[
            jax.ShapeDtypeStruct((B // TC_IMGS, 1, 1), jnp.float32),
            jax.ShapeDtypeStruct((B // TC_IMGS, 1, 1), jnp.int32),
        ],
    )(p, t, m)


def _finish_body(nsc_ref, csc_ref, ntc_ref, ctc_ref, out_ref):
    s = jnp.sum(nsc_ref[...]) + jnp.sum(ntc_ref[...])
    c = jnp.sum(csc_ref[...]) + jnp.sum(ctc_ref[...])
    out_ref[...] = (s / (2.0 * c.astype(jnp.float32))).reshape(1, 1)


def kernel(prediction, target, mask):
    num_sc, cnt_sc = _sc_partials(prediction, target, mask)
    num_tc, cnt_tc = _tc_partials(prediction, target, mask)
    out = pl.pallas_call(
        _finish_body,
        out_shape=jax.ShapeDtypeStruct((1, 1), jnp.float32),
    )(num_sc, cnt_sc, num_tc, cnt_tc)
    return out[0, 0]


# speedup vs baseline: 3.4354x; 1.0002x over previous
import functools

import jax
import jax.numpy as jnp
from jax import lax
from jax.experimental import pallas as pl
from jax.experimental.pallas import tpu as pltpu
from jax.experimental.pallas import tpu_sc as plsc

NC = 2
NS = 16
L = 16
NW = NC * NS

B, H, W = 32, 384, 384
TC_ROWS = 264
SC_ROW0 = TC_ROWS
NCHUNK = 5
CH_ROWS = (H - SC_ROW0) // NCHUNK
TC_IMGS = 8
VPR = W // L
NVEC = CH_ROWS * VPR
UNIT = 8


def _sc_partials(p, t, m):
    mesh = plsc.VectorSubcoreMesh(core_axis_name="c", subcore_axis_name="s")

    @functools.partial(
        pl.kernel,
        mesh=mesh,
        out_type=(
            jax.ShapeDtypeStruct((NW, L), jnp.float32),
            jax.ShapeDtypeStruct((NW, L), jnp.int32),
        ),
        scratch_types=[
            pltpu.VMEM((2, CH_ROWS, W), jnp.float32),
            pltpu.VMEM((2, CH_ROWS, W), jnp.float32),
            pltpu.VMEM((2, CH_ROWS, W), jnp.int32),
            pltpu.VMEM((L,), jnp.float32),
            pltpu.VMEM((L,), jnp.int32),
            pltpu.SemaphoreType.DMA,
            pltpu.SemaphoreType.DMA,
        ],
    )
    def k(p_hbm, t_hbm, m_hbm, num_hbm, cnt_hbm,
          p_v, t_v, m_v, num_v, cnt_v, sem0, sem1):
        wid = lax.axis_index("s") * NC + lax.axis_index("c")
        sems = (sem0, sem1)

        def issue(ci):
            slot = ci % 2
            sl = pl.ds(SC_ROW0 + ci * CH_ROWS, CH_ROWS)
            return (
                pltpu.async_copy(p_hbm.at[wid, sl], p_v.at[slot], sems[slot]),
                pltpu.async_copy(t_hbm.at[wid, sl], t_v.at[slot], sems[slot]),
                pltpu.async_copy(m_hbm.at[wid, sl], m_v.at[slot], sems[slot]),
            )

        def compute(slot, acc, cnt):
            pr, tr, mr = p_v.at[slot], t_v.at[slot], m_v.at[slot]
            zero = jnp.zeros((L,), jnp.float32)
            zeroi = jnp.zeros((L,), jnp.int32)
            carry0 = (acc,) + (zero,) * (UNIT - 1) + (cnt,) + (zeroi,) * (UNIT - 1)

            @plsc.parallel_loop(0, NVEC, step=UNIT, unroll=1, carry=carry0)
            def body(i, c):
                a = list(c[:UNIT])
                n = list(c[UNIT:])
                r = i // VPR
                c0 = (i - r * VPR) * L
                for u in range(UNIT):
                    sl = pl.ds(c0 + u * L, L)
                    ad = jnp.abs(pr[r, sl] - tr[r, sl])
                    mv = mr[r, sl]
                    a[u] = a[u] + jnp.where(mv != 0, ad, 0.0)
                    n[u] = n[u] + mv
                return tuple(a) + tuple(n)

            c = body

            def tree_sum(vals):
                vals = list(vals)
                while len(vals) > 1:
                    vals = [vals[i] + vals[i + 1] for i in range(0, len(vals) - 1, 2)] + (
                        [vals[-1]] if len(vals) % 2 else [])
                return vals[0]

            return tree_sum(c[:UNIT]), tree_sum(c[UNIT:])

        acc = jnp.zeros((L,), jnp.float32)
        cnt = jnp.zeros((L,), jnp.int32)
        handles = {0: issue(0)}
        for ci in range(NCHUNK):
            if ci + 1 < NCHUNK:
                handles[ci + 1] = issue(ci + 1)
            for h in handles.pop(ci):
                h.wait()
            acc, cnt = compute(ci % 2, acc, cnt)
        num_v[...] = acc
        cnt_v[...] = cnt
        pltpu.sync_copy(num_v, num_hbm.at[wid])
        pltpu.sync_copy(cnt_v, cnt_hbm.at[wid])

    return k(p, t, m)


def _tc_body(p_ref, t_ref, m_ref, num_ref, cnt_ref):
    pv = p_ref[...]
    tv = t_ref[...]
    mv = m_ref[...]
    ad = jnp.abs(pv - tv)
    num_ref[...] = jnp.sum(jnp.where(mv != 0, ad, 0.0)).reshape(1, 1, 1)
    cnt_ref[...] = jnp.sum(mv).reshape(1, 1, 1)


def _tc_partials(p, t, m):
    in_spec = pl.BlockSpec((TC_IMGS, TC_ROWS, W), lambda i: (i, 0, 0))
    out_spec = pl.BlockSpec((1, 1, 1), lambda i: (i, 0, 0))
    return pl.pallas_call(
        _tc_body,
        grid=(B // TC_IMGS,),
        in_specs=[in_spec, in_spec, in_spec],
        out_specs=[out_spec, out_spec],
        out_shape=[
            jax.ShapeDtypeStruct((B // TC_IMGS, 1, 1), jnp.float32),
            jax.ShapeDtypeStruct((B // TC_IMGS, 1, 1), jnp.int32),
        ],
    )(p, t, m)


def _finish_body(nsc_ref, csc_ref, ntc_ref, ctc_ref, out_ref):
    s = jnp.sum(nsc_ref[...]) + jnp.sum(ntc_ref[...])
    c = jnp.sum(csc_ref[...]) + jnp.sum(ctc_ref[...])
    out_ref[...] = (s / (2.0 * c.astype(jnp.float32))).reshape(1, 1)


def kernel(prediction, target, mask):
    num_sc, cnt_sc = _sc_partials(prediction, target, mask)
    num_tc, cnt_tc = _tc_partials(prediction, target, mask)
    out = pl.pallas_call(
        _finish_body,
        out_shape=jax.ShapeDtypeStruct((1, 1), jnp.float32),
    )(num_sc, cnt_sc, num_tc, cnt_tc)
    return out[0, 0]
